# Initial kernel scaffold; baseline (speedup 1.0000x reference)
import jax
import jax.numpy as jnp
from jax.experimental import pallas as pl


def kernel(x, edge_index, batch, W, b, gamma, beta):
    raise NotImplementedError("write your pallas kernel here")


# speedup vs baseline: 7.6746x; 7.6746x over previous
import functools

import jax
import jax.numpy as jnp
from jax import lax
from jax.experimental import pallas as pl
from jax.experimental.pallas import tpu as pltpu
from jax.experimental.pallas import tpu_sc as plsc

_N = 10000
_E = 320000
_D = 128
_H = _D // 2
_EPS = 1e-7

_NSUB = 16
_CHUNK = 80
_EPT = _E // _NSUB
_CPT = _EPT // _CHUNK
_RPT = 640
_RLAST = _N - 15 * _RPT

_BN = 2000


@functools.cache
def _seg_kernel():
    mesh = plsc.VectorSubcoreMesh(core_axis_name="c", subcore_axis_name="s")

    @functools.partial(
        pl.kernel,
        out_type=(jax.ShapeDtypeStruct((_N, _H), jnp.float32),) * 4,
        mesh=mesh,
        scratch_types=[
            pltpu.VMEM((_CPT, _CHUNK), jnp.int32),
            pltpu.VMEM((_CPT, _CHUNK), jnp.int32),
            pltpu.VMEM((_CHUNK, _H), jnp.float32),
            pltpu.VMEM_SHARED((_N, _H), jnp.float32),
            pltpu.SemaphoreType.DMA,
        ],
        compiler_params=pltpu.CompilerParams(use_tc_tiling_on_sc=False),
    )
    def seg_kernel(t0a_hbm, t0b_hbm, t1a_hbm, t1b_hbm, src_hbm, dst_hbm,
                   z_hbm, s1a_hbm, s1b_hbm, s2a_hbm, s2b_hbm,
                   idx_s, idx_d, rows, acc, sem):
        cid = lax.axis_index("c")
        sid = lax.axis_index("s")

        pltpu.sync_copy(src_hbm.at[sid], idx_s)
        pltpu.sync_copy(dst_hbm.at[sid], idx_d)

        def do_pass(tab_hbm, out_hbm):
            @pl.when(sid < 15)
            def _():
                pltpu.sync_copy(z_hbm.at[pl.ds(sid * _RPT, _RPT)],
                                acc.at[pl.ds(sid * _RPT, _RPT)])

            @pl.when(sid == 15)
            def _():
                pltpu.sync_copy(z_hbm.at[pl.ds(15 * _RPT, _RLAST)],
                                acc.at[pl.ds(15 * _RPT, _RLAST)])

            plsc.subcore_barrier()

            def body(ci, carry):
                pltpu.async_copy(tab_hbm.at[idx_s.at[ci]], rows, sem).wait()
                pltpu.sync_copy(rows, acc.at[idx_d.at[ci]], add=True)
                return carry

            lax.fori_loop(0, _CPT, body, 0)
            plsc.subcore_barrier()

            @pl.when(sid < 15)
            def _():
                pltpu.sync_copy(acc.at[pl.ds(sid * _RPT, _RPT)],
                                out_hbm.at[pl.ds(sid * _RPT, _RPT)])

            @pl.when(sid == 15)
            def _():
                pltpu.sync_copy(acc.at[pl.ds(15 * _RPT, _RLAST)],
                                out_hbm.at[pl.ds(15 * _RPT, _RLAST)])

        @pl.when(cid == 0)
        def _():
            do_pass(t0a_hbm, s1a_hbm)
            do_pass(t1a_hbm, s2a_hbm)

        @pl.when(cid == 1)
        def _():
            do_pass(t0b_hbm, s1b_hbm)
            do_pass(t1b_hbm, s2b_hbm)

    return seg_kernel


def _sc_segsum(tabs, src_rows, dst_rows, zeros):
    t0a, t0b, t1a, t1b = tabs
    return _seg_kernel()(t0a, t0b, t1a, t1b, src_rows, dst_rows, zeros)


def _layernorm(h, g, b):
    mu = jnp.mean(h, axis=-1, keepdims=True)
    var = jnp.mean((h - mu) * (h - mu), axis=-1, keepdims=True)
    return (h - mu) / jnp.sqrt(var + 1e-5) * g + b


def _tables(z):
    g = z + _EPS
    e = jnp.exp(g)
    t0, t1 = e, g * e
    return t0[:, :_H], t0[:, _H:], t1[:, :_H], t1[:, _H:]


def _agg(s1a, s1b, s2a, s2b):
    s1 = jnp.concatenate([s1a, s1b], axis=1)
    s2 = jnp.concatenate([s2a, s2b], axis=1)
    return jnp.where(s1 > 0.0, s2 / (s1 + 1e-16), 0.0)


_BS = pl.BlockSpec((_BN, _D), lambda i: (i, 0))
_BSH = pl.BlockSpec((_BN, _H), lambda i: (i, 0))
_WS = pl.BlockSpec((_D, _D), lambda i: (0, 0))
_VS = pl.BlockSpec((1, _D), lambda i: (0, 0))
_T4 = (jax.ShapeDtypeStruct((_N, _H), jnp.float32),) * 4


def _prep_body(x_ref, t0a, t0b, t1a, t1b):
    z = jnp.maximum(x_ref[...], 0.0)
    t0a[...], t0b[...], t1a[...], t1b[...] = _tables(z)


def _prep(x):
    return pl.pallas_call(
        _prep_body,
        grid=(_N // _BN,),
        in_specs=[_BS],
        out_specs=[_BSH] * 4,
        out_shape=_T4,
    )(x)


def _update(S4, zres, carry, Wl, bl, gl, betal):
    has_carry = carry is not None

    def body(*refs):
        if has_carry:
            s1a, s1b, s2a, s2b, z, c, w, b2, g2, be2 = refs[:10]
            outs = refs[10:]
        else:
            s1a, s1b, s2a, s2b, z, w, b2, g2, be2 = refs[:9]
            outs = refs[9:]
        h_ref, z_ref, t0a, t0b, t1a, t1b = outs
        a = _agg(s1a[...], s1b[...], s2a[...], s2b[...])
        h = jnp.dot(z[...] + a, w[...], preferred_element_type=jnp.float32)
        h = h + b2[...]
        if has_carry:
            h = h + c[...]
        zn = jnp.maximum(_layernorm(h, g2[...], be2[...]), 0.0)
        t0a[...], t0b[...], t1a[...], t1b[...] = _tables(zn)
        h_ref[...] = h
        z_ref[...] = zn

    in_specs = ([_BSH] * 4 + [_BS] + ([_BS] if has_carry else [])
                + [_WS, _VS, _VS, _VS])
    args = (list(S4) + [zres] + ([carry] if has_carry else [])
            + [Wl, bl, gl, betal])
    return pl.pallas_call(
        body,
        grid=(_N // _BN,),
        in_specs=in_specs,
        out_specs=[_BS, _BS] + [_BSH] * 4,
        out_shape=(jax.ShapeDtypeStruct((_N, _D), jnp.float32),) * 2 + _T4,
    )(*args)


def _final_body(s1a, s1b, s2a, s2b, z, c, w, b2, g2, be2, out_ref):
    a = _agg(s1a[...], s1b[...], s2a[...], s2b[...])
    h = jnp.dot(z[...] + a, w[...], preferred_element_type=jnp.float32)
    h = h + b2[...] + c[...]
    out_ref[...] = _layernorm(h, g2[...], be2[...])


def _final(S4, zres, carry, Wl, bl, gl, betal):
    return pl.pallas_call(
        _final_body,
        grid=(_N // _BN,),
        in_specs=[_BSH] * 4 + [_BS, _BS, _WS, _VS, _VS, _VS],
        out_specs=_BS,
        out_shape=jax.ShapeDtypeStruct((_N, _D), jnp.float32),
    )(*S4, zres, carry, Wl, bl, gl, betal)


def kernel(x, edge_index, batch, W, b, gamma, beta):
    del batch
    src = edge_index[0].reshape(_NSUB, _CPT, _CHUNK)
    dst = edge_index[1].reshape(_NSUB, _CPT, _CHUNK)
    zeros = jnp.zeros((_N, _H), jnp.float32)
    b2 = b[:, None, :]
    g2 = gamma[:, None, :]
    be2 = beta[:, None, :]

    tabs = _prep(x)
    S4 = _sc_segsum(tabs, src, dst, zeros)
    h1, z1, *tabs = _update(S4, x, None, W[0], b2[0], g2[0], be2[0])
    S4 = _sc_segsum(tabs, src, dst, zeros)
    h2, z2, *tabs = _update(S4, z1, h1, W[1], b2[1], g2[1], be2[1])
    S4 = _sc_segsum(tabs, src, dst, zeros)
    return _final(S4, z2, h2, W[2], b2[2], g2[2], be2[2])


# speedup vs baseline: 11.9743x; 1.5602x over previous
import functools

import jax
import jax.numpy as jnp
from jax import lax
from jax.experimental import pallas as pl
from jax.experimental.pallas import tpu as pltpu
from jax.experimental.pallas import tpu_sc as plsc

_N = 10000
_E = 320000
_D = 128
_H = _D // 2
_EPS = 1e-7

_NSUB = 16
_CHUNK = 125
_EPT = _E // _NSUB
_CPT = _EPT // _CHUNK
_RPT = 640
_RLAST = _N - 15 * _RPT

_BN = 2000


@functools.cache
def _seg_kernel():
    mesh = plsc.VectorSubcoreMesh(core_axis_name="c", subcore_axis_name="s")

    @functools.partial(
        pl.kernel,
        out_type=(jax.ShapeDtypeStruct((_N, _H), jnp.float32),) * 4,
        mesh=mesh,
        scratch_types=[
            pltpu.VMEM((_CPT, _CHUNK), jnp.int32),
            pltpu.VMEM((_CPT, _CHUNK), jnp.int32),
            pltpu.VMEM((_CHUNK, _H), jnp.float32),
            pltpu.VMEM((_CHUNK, _H), jnp.float32),
            pltpu.VMEM_SHARED((_N, _H), jnp.float32),
            pltpu.SemaphoreType.DMA,
            pltpu.SemaphoreType.DMA,
        ],
        compiler_params=pltpu.CompilerParams(use_tc_tiling_on_sc=False),
    )
    def seg_kernel(t0a_hbm, t0b_hbm, t1a_hbm, t1b_hbm, src_hbm, dst_hbm,
                   z_hbm, s1a_hbm, s1b_hbm, s2a_hbm, s2b_hbm,
                   idx_s, idx_d, rows0, rows1, acc, sem0, sem1):
        cid = lax.axis_index("c")
        sid = lax.axis_index("s")

        pltpu.sync_copy(src_hbm.at[sid], idx_s)
        pltpu.sync_copy(dst_hbm.at[sid], idx_d)

        def do_pass(tab_hbm, out_hbm):
            @pl.when(sid < 15)
            def _():
                pltpu.sync_copy(z_hbm.at[pl.ds(sid * _RPT, _RPT)],
                                acc.at[pl.ds(sid * _RPT, _RPT)])

            @pl.when(sid == 15)
            def _():
                pltpu.sync_copy(z_hbm.at[pl.ds(15 * _RPT, _RLAST)],
                                acc.at[pl.ds(15 * _RPT, _RLAST)])

            plsc.subcore_barrier()

            g0 = pltpu.async_copy(tab_hbm.at[idx_s.at[0]], rows0, sem0)

            def body(i, carry):
                c0 = 2 * i
                pltpu.make_async_copy(tab_hbm.at[idx_s.at[c0]], rows0,
                                      sem0).wait()
                pltpu.async_copy(tab_hbm.at[idx_s.at[c0 + 1]], rows1, sem1)
                pltpu.sync_copy(rows0, acc.at[idx_d.at[c0]], add=True)
                pltpu.make_async_copy(tab_hbm.at[idx_s.at[c0 + 1]], rows1,
                                      sem1).wait()
                nxt = jnp.minimum(c0 + 2, _CPT - 1)
                pltpu.async_copy(tab_hbm.at[idx_s.at[nxt]], rows0, sem0)
                pltpu.sync_copy(rows1, acc.at[idx_d.at[c0 + 1]], add=True)
                return carry

            lax.fori_loop(0, _CPT // 2, body, 0)
            pltpu.make_async_copy(tab_hbm.at[idx_s.at[_CPT - 1]], rows0,
                                  sem0).wait()
            plsc.subcore_barrier()

            @pl.when(sid < 15)
            def _():
                pltpu.sync_copy(acc.at[pl.ds(sid * _RPT, _RPT)],
                                out_hbm.at[pl.ds(sid * _RPT, _RPT)])

            @pl.when(sid == 15)
            def _():
                pltpu.sync_copy(acc.at[pl.ds(15 * _RPT, _RLAST)],
                                out_hbm.at[pl.ds(15 * _RPT, _RLAST)])

        @pl.when(cid == 0)
        def _():
            do_pass(t0a_hbm, s1a_hbm)
            do_pass(t1a_hbm, s2a_hbm)

        @pl.when(cid == 1)
        def _():
            do_pass(t0b_hbm, s1b_hbm)
            do_pass(t1b_hbm, s2b_hbm)

    return seg_kernel


def _sc_segsum(tabs, src_rows, dst_rows, zeros):
    t0a, t0b, t1a, t1b = tabs
    return _seg_kernel()(t0a, t0b, t1a, t1b, src_rows, dst_rows, zeros)


def _layernorm(h, g, b):
    mu = jnp.mean(h, axis=-1, keepdims=True)
    var = jnp.mean((h - mu) * (h - mu), axis=-1, keepdims=True)
    return (h - mu) / jnp.sqrt(var + 1e-5) * g + b


def _tables(z):
    g = z + _EPS
    e = jnp.exp(g)
    t0, t1 = e, g * e
    return t0[:, :_H], t0[:, _H:], t1[:, :_H], t1[:, _H:]


def _agg(s1a, s1b, s2a, s2b):
    s1 = jnp.concatenate([s1a, s1b], axis=1)
    s2 = jnp.concatenate([s2a, s2b], axis=1)
    return jnp.where(s1 > 0.0, s2 / (s1 + 1e-16), 0.0)


_BS = pl.BlockSpec((_BN, _D), lambda i: (i, 0))
_BSH = pl.BlockSpec((_BN, _H), lambda i: (i, 0))
_WS = pl.BlockSpec((_D, _D), lambda i: (0, 0))
_VS = pl.BlockSpec((1, _D), lambda i: (0, 0))
_T4 = (jax.ShapeDtypeStruct((_N, _H), jnp.float32),) * 4


def _prep_body(x_ref, t0a, t0b, t1a, t1b):
    z = jnp.maximum(x_ref[...], 0.0)
    t0a[...], t0b[...], t1a[...], t1b[...] = _tables(z)


def _prep(x):
    return pl.pallas_call(
        _prep_body,
        grid=(_N // _BN,),
        in_specs=[_BS],
        out_specs=[_BSH] * 4,
        out_shape=_T4,
    )(x)


def _update(S4, zres, carry, Wl, bl, gl, betal):
    has_carry = carry is not None

    def body(*refs):
        if has_carry:
            s1a, s1b, s2a, s2b, z, c, w, b2, g2, be2 = refs[:10]
            outs = refs[10:]
        else:
            s1a, s1b, s2a, s2b, z, w, b2, g2, be2 = refs[:9]
            outs = refs[9:]
        h_ref, z_ref, t0a, t0b, t1a, t1b = outs
        a = _agg(s1a[...], s1b[...], s2a[...], s2b[...])
        h = jnp.dot(z[...] + a, w[...], preferred_element_type=jnp.float32)
        h = h + b2[...]
        if has_carry:
            h = h + c[...]
        zn = jnp.maximum(_layernorm(h, g2[...], be2[...]), 0.0)
        t0a[...], t0b[...], t1a[...], t1b[...] = _tables(zn)
        h_ref[...] = h
        z_ref[...] = zn

    in_specs = ([_BSH] * 4 + [_BS] + ([_BS] if has_carry else [])
                + [_WS, _VS, _VS, _VS])
    args = (list(S4) + [zres] + ([carry] if has_carry else [])
            + [Wl, bl, gl, betal])
    return pl.pallas_call(
        body,
        grid=(_N // _BN,),
        in_specs=in_specs,
        out_specs=[_BS, _BS] + [_BSH] * 4,
        out_shape=(jax.ShapeDtypeStruct((_N, _D), jnp.float32),) * 2 + _T4,
    )(*args)


def _final_body(s1a, s1b, s2a, s2b, z, c, w, b2, g2, be2, out_ref):
    a = _agg(s1a[...], s1b[...], s2a[...], s2b[...])
    h = jnp.dot(z[...] + a, w[...], preferred_element_type=jnp.float32)
    h = h + b2[...] + c[...]
    out_ref[...] = _layernorm(h, g2[...], be2[...])


def _final(S4, zres, carry, Wl, bl, gl, betal):
    return pl.pallas_call(
        _final_body,
        grid=(_N // _BN,),
        in_specs=[_BSH] * 4 + [_BS, _BS, _WS, _VS, _VS, _VS],
        out_specs=_BS,
        out_shape=jax.ShapeDtypeStruct((_N, _D), jnp.float32),
    )(*S4, zres, carry, Wl, bl, gl, betal)


def kernel(x, edge_index, batch, W, b, gamma, beta):
    del batch
    src = edge_index[0].reshape(_NSUB, _CPT, _CHUNK)
    dst = edge_index[1].reshape(_NSUB, _CPT, _CHUNK)
    zeros = jnp.zeros((_N, _H), jnp.float32)
    b2 = b[:, None, :]
    g2 = gamma[:, None, :]
    be2 = beta[:, None, :]

    tabs = _prep(x)
    S4 = _sc_segsum(tabs, src, dst, zeros)
    h1, z1, *tabs = _update(S4, x, None, W[0], b2[0], g2[0], be2[0])
    S4 = _sc_segsum(tabs, src, dst, zeros)
    h2, z2, *tabs = _update(S4, z1, h1, W[1], b2[1], g2[1], be2[1])
    S4 = _sc_segsum(tabs, src, dst, zeros)
    return _final(S4, z2, h2, W[2], b2[2], g2[2], be2[2])


# speedup vs baseline: 18.1317x; 1.5142x over previous
import functools

import jax
import jax.numpy as jnp
from jax import lax
from jax.experimental import pallas as pl
from jax.experimental.pallas import tpu as pltpu
from jax.experimental.pallas import tpu_sc as plsc

_N = 10000
_E = 320000
_D = 128
_H = _D // 2
_EPS = 1e-7

_NSUB = 16
_CHUNK = 125
_EPT = _E // _NSUB
_CPT = _EPT // _CHUNK
_K = 2
_NB = _CPT // _K
_RPT = 640
_RLAST = _N - 15 * _RPT

_BN = 2000


@functools.cache
def _seg_kernel():
    mesh = plsc.VectorSubcoreMesh(core_axis_name="c", subcore_axis_name="s")

    @functools.partial(
        pl.kernel,
        out_type=(jax.ShapeDtypeStruct((_N, _H), jnp.float32),) * 4,
        mesh=mesh,
        scratch_types=[
            pltpu.VMEM((_CPT, _CHUNK), jnp.int32),
            pltpu.VMEM((_CPT, _CHUNK), jnp.int32),
            pltpu.VMEM((3 * _K, _CHUNK, _H), jnp.float32),
            pltpu.VMEM_SHARED((_N, _H), jnp.float32),
            pltpu.SemaphoreType.DMA,
            pltpu.SemaphoreType.DMA,
            pltpu.SemaphoreType.DMA,
            pltpu.SemaphoreType.DMA,
            pltpu.SemaphoreType.DMA,
            pltpu.SemaphoreType.DMA,
        ],
        compiler_params=pltpu.CompilerParams(use_tc_tiling_on_sc=False),
    )
    def seg_kernel(t0a_hbm, t0b_hbm, t1a_hbm, t1b_hbm, src_hbm, dst_hbm,
                   z_hbm, s1a_hbm, s1b_hbm, s2a_hbm, s2b_hbm,
                   idx_s, idx_d, rows, acc,
                   semg0, semg1, semg2, sems0, sems1, sems2):
        cid = lax.axis_index("c")
        sid = lax.axis_index("s")

        pltpu.sync_copy(src_hbm.at[sid], idx_s)
        pltpu.sync_copy(dst_hbm.at[sid], idx_d)

        def do_pass(tab_hbm, out_hbm):
            @pl.when(sid < 15)
            def _():
                pltpu.sync_copy(z_hbm.at[pl.ds(sid * _RPT, _RPT)],
                                acc.at[pl.ds(sid * _RPT, _RPT)])

            @pl.when(sid == 15)
            def _():
                pltpu.sync_copy(z_hbm.at[pl.ds(15 * _RPT, _RLAST)],
                                acc.at[pl.ds(15 * _RPT, _RLAST)])

            plsc.subcore_barrier()

            semg = (semg0, semg1, semg2)
            sems = (sems0, sems1, sems2)

            def g_issue(n, grp):
                for k in range(_K):
                    c = jnp.minimum(n * _K + k, _CPT - 1)
                    pltpu.async_copy(tab_hbm.at[idx_s.at[c]],
                                     rows.at[_K * grp + k], semg[grp])

            def g_wait(n, grp):
                for k in range(_K):
                    c = jnp.minimum(n * _K + k, _CPT - 1)
                    pltpu.make_async_copy(tab_hbm.at[idx_s.at[c]],
                                          rows.at[_K * grp + k],
                                          semg[grp]).wait()

            def s_issue(n, grp):
                for k in range(_K):
                    c = n * _K + k
                    pltpu.async_copy(rows.at[_K * grp + k],
                                     acc.at[idx_d.at[c]], sems[grp],
                                     add=True)

            def s_wait(n, grp):
                for k in range(_K):
                    c = n * _K + k
                    pltpu.make_async_copy(rows.at[_K * grp + k],
                                          acc.at[idx_d.at[c]],
                                          sems[grp]).wait()

            g_issue(0, 0)
            g_issue(1, 1)
            g_wait(0, 0)
            s_issue(0, 0)
            g_issue(2, 2)
            g_wait(1, 1)
            s_issue(1, 1)

            def body(i, carry):
                for j in range(3):
                    n = 3 * i + 2 + j
                    grp = (2 + j) % 3
                    s_wait(n - 2, j)
                    g_issue(n + 1, j)
                    g_wait(n, grp)
                    s_issue(n, grp)
                return carry

            lax.fori_loop(0, (_NB - 2) // 3, body, 0)
            s_wait(_NB - 2, (_NB - 2) % 3)
            s_wait(_NB - 1, (_NB - 1) % 3)
            g_wait(_NB, _NB % 3)
            plsc.subcore_barrier()

            @pl.when(sid < 15)
            def _():
                pltpu.sync_copy(acc.at[pl.ds(sid * _RPT, _RPT)],
                                out_hbm.at[pl.ds(sid * _RPT, _RPT)])

            @pl.when(sid == 15)
            def _():
                pltpu.sync_copy(acc.at[pl.ds(15 * _RPT, _RLAST)],
                                out_hbm.at[pl.ds(15 * _RPT, _RLAST)])

        @pl.when(cid == 0)
        def _():
            do_pass(t0a_hbm, s1a_hbm)
            do_pass(t1a_hbm, s2a_hbm)

        @pl.when(cid == 1)
        def _():
            do_pass(t0b_hbm, s1b_hbm)
            do_pass(t1b_hbm, s2b_hbm)

    return seg_kernel


def _sc_segsum(tabs, src_rows, dst_rows, zeros):
    t0a, t0b, t1a, t1b = tabs
    return _seg_kernel()(t0a, t0b, t1a, t1b, src_rows, dst_rows, zeros)


def _layernorm(h, g, b):
    mu = jnp.mean(h, axis=-1, keepdims=True)
    var = jnp.mean((h - mu) * (h - mu), axis=-1, keepdims=True)
    return (h - mu) / jnp.sqrt(var + 1e-5) * g + b


def _tables(z):
    g = z + _EPS
    e = jnp.exp(g)
    t0, t1 = e, g * e
    return t0[:, :_H], t0[:, _H:], t1[:, :_H], t1[:, _H:]


def _agg(s1a, s1b, s2a, s2b):
    s1 = jnp.concatenate([s1a, s1b], axis=1)
    s2 = jnp.concatenate([s2a, s2b], axis=1)
    return jnp.where(s1 > 0.0, s2 / (s1 + 1e-16), 0.0)


_BS = pl.BlockSpec((_BN, _D), lambda i: (i, 0))
_BSH = pl.BlockSpec((_BN, _H), lambda i: (i, 0))
_WS = pl.BlockSpec((_D, _D), lambda i: (0, 0))
_VS = pl.BlockSpec((1, _D), lambda i: (0, 0))
_T4 = (jax.ShapeDtypeStruct((_N, _H), jnp.float32),) * 4


def _prep_body(x_ref, t0a, t0b, t1a, t1b):
    z = jnp.maximum(x_ref[...], 0.0)
    t0a[...], t0b[...], t1a[...], t1b[...] = _tables(z)


def _prep(x):
    return pl.pallas_call(
        _prep_body,
        grid=(_N // _BN,),
        in_specs=[_BS],
        out_specs=[_BSH] * 4,
        out_shape=_T4,
    )(x)


def _update(S4, zres, carry, Wl, bl, gl, betal):
    has_carry = carry is not None

    def body(*refs):
        if has_carry:
            s1a, s1b, s2a, s2b, z, c, w, b2, g2, be2 = refs[:10]
            outs = refs[10:]
        else:
            s1a, s1b, s2a, s2b, z, w, b2, g2, be2 = refs[:9]
            outs = refs[9:]
        h_ref, z_ref, t0a, t0b, t1a, t1b = outs
        a = _agg(s1a[...], s1b[...], s2a[...], s2b[...])
        h = jnp.dot(z[...] + a, w[...], preferred_element_type=jnp.float32)
        h = h + b2[...]
        if has_carry:
            h = h + c[...]
        zn = jnp.maximum(_layernorm(h, g2[...], be2[...]), 0.0)
        t0a[...], t0b[...], t1a[...], t1b[...] = _tables(zn)
        h_ref[...] = h
        z_ref[...] = zn

    in_specs = ([_BSH] * 4 + [_BS] + ([_BS] if has_carry else [])
                + [_WS, _VS, _VS, _VS])
    args = (list(S4) + [zres] + ([carry] if has_carry else [])
            + [Wl, bl, gl, betal])
    return pl.pallas_call(
        body,
        grid=(_N // _BN,),
        in_specs=in_specs,
        out_specs=[_BS, _BS] + [_BSH] * 4,
        out_shape=(jax.ShapeDtypeStruct((_N, _D), jnp.float32),) * 2 + _T4,
    )(*args)


def _final_body(s1a, s1b, s2a, s2b, z, c, w, b2, g2, be2, out_ref):
    a = _agg(s1a[...], s1b[...], s2a[...], s2b[...])
    h = jnp.dot(z[...] + a, w[...], preferred_element_type=jnp.float32)
    h = h + b2[...] + c[...]
    out_ref[...] = _layernorm(h, g2[...], be2[...])


def _final(S4, zres, carry, Wl, bl, gl, betal):
    return pl.pallas_call(
        _final_body,
        grid=(_N // _BN,),
        in_specs=[_BSH] * 4 + [_BS, _BS, _WS, _VS, _VS, _VS],
        out_specs=_BS,
        out_shape=jax.ShapeDtypeStruct((_N, _D), jnp.float32),
    )(*S4, zres, carry, Wl, bl, gl, betal)


def kernel(x, edge_index, batch, W, b, gamma, beta):
    del batch
    src = edge_index[0].reshape(_NSUB, _CPT, _CHUNK)
    dst = edge_index[1].reshape(_NSUB, _CPT, _CHUNK)
    zeros = jnp.zeros((_N, _H), jnp.float32)
    b2 = b[:, None, :]
    g2 = gamma[:, None, :]
    be2 = beta[:, None, :]

    tabs = _prep(x)
    S4 = _sc_segsum(tabs, src, dst, zeros)
    h1, z1, *tabs = _update(S4, x, None, W[0], b2[0], g2[0], be2[0])
    S4 = _sc_segsum(tabs, src, dst, zeros)
    h2, z2, *tabs = _update(S4, z1, h1, W[1], b2[1], g2[1], be2[1])
    S4 = _sc_segsum(tabs, src, dst, zeros)
    return _final(S4, z2, h2, W[2], b2[2], g2[2], be2[2])


# speedup vs baseline: 19.9004x; 1.0975x over previous
import functools

import jax
import jax.numpy as jnp
from jax import lax
from jax.experimental import pallas as pl
from jax.experimental.pallas import tpu as pltpu
from jax.experimental.pallas import tpu_sc as plsc

_N = 10000
_E = 320000
_D = 128
_H = _D // 2
_EPS = 1e-7

_NSUB = 16
_CHUNK = 125
_EPT = _E // _NSUB
_CPT = _EPT // _CHUNK
_K = 2
_NB = _CPT // _K
_RPT = 640
_RLAST = _N - 15 * _RPT

_BN = 2000


@functools.cache
def _seg_kernel():
    mesh = plsc.VectorSubcoreMesh(core_axis_name="c", subcore_axis_name="s")

    @functools.partial(
        pl.kernel,
        out_type=(jax.ShapeDtypeStruct((_N, _H), jnp.float32),) * 4,
        mesh=mesh,
        scratch_types=[
            pltpu.VMEM((_CPT, _CHUNK), jnp.int32),
            pltpu.VMEM((_CPT, _CHUNK), jnp.int32),
            pltpu.VMEM((3 * _K, _CHUNK, _H), jnp.float32),
            pltpu.VMEM_SHARED((_N, _H), jnp.float32),
            pltpu.SemaphoreType.DMA,
            pltpu.SemaphoreType.DMA,
            pltpu.SemaphoreType.DMA,
            pltpu.SemaphoreType.DMA,
            pltpu.SemaphoreType.DMA,
            pltpu.SemaphoreType.DMA,
        ],
        compiler_params=pltpu.CompilerParams(use_tc_tiling_on_sc=False),
    )
    def seg_kernel(t0a_hbm, t0b_hbm, t1a_hbm, t1b_hbm, src_hbm, dst_hbm,
                   z_hbm, s1a_hbm, s1b_hbm, s2a_hbm, s2b_hbm,
                   idx_s, idx_d, rows, acc,
                   semg0, semg1, semg2, sems0, sems1, sems2):
        cid = lax.axis_index("c")
        sid = lax.axis_index("s")

        pltpu.sync_copy(src_hbm.at[sid], idx_s)
        pltpu.sync_copy(dst_hbm.at[sid], idx_d)

        def do_pass(tab_hbm, out_hbm):
            @pl.when(sid < 15)
            def _():
                pltpu.sync_copy(z_hbm.at[pl.ds(sid * _RPT, _RPT)],
                                acc.at[pl.ds(sid * _RPT, _RPT)])

            @pl.when(sid == 15)
            def _():
                pltpu.sync_copy(z_hbm.at[pl.ds(15 * _RPT, _RLAST)],
                                acc.at[pl.ds(15 * _RPT, _RLAST)])

            plsc.subcore_barrier()

            semg = (semg0, semg1, semg2)
            sems = (sems0, sems1, sems2)

            def g_issue(n, grp):
                for k in range(_K):
                    c = jnp.minimum(n * _K + k, _CPT - 1)
                    pltpu.async_copy(tab_hbm.at[idx_s.at[c]],
                                     rows.at[_K * grp + k], semg[grp])

            def g_wait(n, grp):
                for k in range(_K):
                    c = jnp.minimum(n * _K + k, _CPT - 1)
                    pltpu.make_async_copy(tab_hbm.at[idx_s.at[c]],
                                          rows.at[_K * grp + k],
                                          semg[grp]).wait()

            def s_issue(n, grp):
                for k in range(_K):
                    c = n * _K + k
                    pltpu.async_copy(rows.at[_K * grp + k],
                                     acc.at[idx_d.at[c]], sems[grp],
                                     add=True)

            def s_wait(n, grp):
                for k in range(_K):
                    c = n * _K + k
                    pltpu.make_async_copy(rows.at[_K * grp + k],
                                          acc.at[idx_d.at[c]],
                                          sems[grp]).wait()

            g_issue(0, 0)
            g_issue(1, 1)
            g_wait(0, 0)
            g_issue(2, 2)
            g_wait(1, 1)

            def body(i, carry):
                for j in range(3):
                    n = 3 * i + 2 + j
                    grp = (2 + j) % 3
                    g_issue(n + 1, j)
                    g_wait(n, grp)
                return carry

            lax.fori_loop(0, (_NB - 2) // 3, body, 0)
            g_wait(_NB, _NB % 3)
            plsc.subcore_barrier()

            @pl.when(sid < 15)
            def _():
                pltpu.sync_copy(acc.at[pl.ds(sid * _RPT, _RPT)],
                                out_hbm.at[pl.ds(sid * _RPT, _RPT)])

            @pl.when(sid == 15)
            def _():
                pltpu.sync_copy(acc.at[pl.ds(15 * _RPT, _RLAST)],
                                out_hbm.at[pl.ds(15 * _RPT, _RLAST)])

        @pl.when(cid == 0)
        def _():
            do_pass(t0a_hbm, s1a_hbm)
            do_pass(t1a_hbm, s2a_hbm)

        @pl.when(cid == 1)
        def _():
            do_pass(t0b_hbm, s1b_hbm)
            do_pass(t1b_hbm, s2b_hbm)

    return seg_kernel


def _sc_segsum(tabs, src_rows, dst_rows, zeros):
    t0a, t0b, t1a, t1b = tabs
    return _seg_kernel()(t0a, t0b, t1a, t1b, src_rows, dst_rows, zeros)


def _layernorm(h, g, b):
    mu = jnp.mean(h, axis=-1, keepdims=True)
    var = jnp.mean((h - mu) * (h - mu), axis=-1, keepdims=True)
    return (h - mu) / jnp.sqrt(var + 1e-5) * g + b


def _tables(z):
    g = z + _EPS
    e = jnp.exp(g)
    t0, t1 = e, g * e
    return t0[:, :_H], t0[:, _H:], t1[:, :_H], t1[:, _H:]


def _agg(s1a, s1b, s2a, s2b):
    s1 = jnp.concatenate([s1a, s1b], axis=1)
    s2 = jnp.concatenate([s2a, s2b], axis=1)
    return jnp.where(s1 > 0.0, s2 / (s1 + 1e-16), 0.0)


_BS = pl.BlockSpec((_BN, _D), lambda i: (i, 0))
_BSH = pl.BlockSpec((_BN, _H), lambda i: (i, 0))
_WS = pl.BlockSpec((_D, _D), lambda i: (0, 0))
_VS = pl.BlockSpec((1, _D), lambda i: (0, 0))
_T4 = (jax.ShapeDtypeStruct((_N, _H), jnp.float32),) * 4


def _prep_body(x_ref, t0a, t0b, t1a, t1b):
    z = jnp.maximum(x_ref[...], 0.0)
    t0a[...], t0b[...], t1a[...], t1b[...] = _tables(z)


def _prep(x):
    return pl.pallas_call(
        _prep_body,
        grid=(_N // _BN,),
        in_specs=[_BS],
        out_specs=[_BSH] * 4,
        out_shape=_T4,
    )(x)


def _update(S4, zres, carry, Wl, bl, gl, betal):
    has_carry = carry is not None

    def body(*refs):
        if has_carry:
            s1a, s1b, s2a, s2b, z, c, w, b2, g2, be2 = refs[:10]
            outs = refs[10:]
        else:
            s1a, s1b, s2a, s2b, z, w, b2, g2, be2 = refs[:9]
            outs = refs[9:]
        h_ref, z_ref, t0a, t0b, t1a, t1b = outs
        a = _agg(s1a[...], s1b[...], s2a[...], s2b[...])
        h = jnp.dot(z[...] + a, w[...], preferred_element_type=jnp.float32)
        h = h + b2[...]
        if has_carry:
            h = h + c[...]
        zn = jnp.maximum(_layernorm(h, g2[...], be2[...]), 0.0)
        t0a[...], t0b[...], t1a[...], t1b[...] = _tables(zn)
        h_ref[...] = h
        z_ref[...] = zn

    in_specs = ([_BSH] * 4 + [_BS] + ([_BS] if has_carry else [])
                + [_WS, _VS, _VS, _VS])
    args = (list(S4) + [zres] + ([carry] if has_carry else [])
            + [Wl, bl, gl, betal])
    return pl.pallas_call(
        body,
        grid=(_N // _BN,),
        in_specs=in_specs,
        out_specs=[_BS, _BS] + [_BSH] * 4,
        out_shape=(jax.ShapeDtypeStruct((_N, _D), jnp.float32),) * 2 + _T4,
    )(*args)


def _final_body(s1a, s1b, s2a, s2b, z, c, w, b2, g2, be2, out_ref):
    a = _agg(s1a[...], s1b[...], s2a[...], s2b[...])
    h = jnp.dot(z[...] + a, w[...], preferred_element_type=jnp.float32)
    h = h + b2[...] + c[...]
    out_ref[...] = _layernorm(h, g2[...], be2[...])


def _final(S4, zres, carry, Wl, bl, gl, betal):
    return pl.pallas_call(
        _final_body,
        grid=(_N // _BN,),
        in_specs=[_BSH] * 4 + [_BS, _BS, _WS, _VS, _VS, _VS],
        out_specs=_BS,
        out_shape=jax.ShapeDtypeStruct((_N, _D), jnp.float32),
    )(*S4, zres, carry, Wl, bl, gl, betal)


def kernel(x, edge_index, batch, W, b, gamma, beta):
    del batch
    src = edge_index[0].reshape(_NSUB, _CPT, _CHUNK)
    dst = edge_index[1].reshape(_NSUB, _CPT, _CHUNK)
    zeros = jnp.zeros((_N, _H), jnp.float32)
    b2 = b[:, None, :]
    g2 = gamma[:, None, :]
    be2 = beta[:, None, :]

    tabs = _prep(x)
    S4 = _sc_segsum(tabs, src, dst, zeros)
    h1, z1, *tabs = _update(S4, x, None, W[0], b2[0], g2[0], be2[0])
    S4 = _sc_segsum(tabs, src, dst, zeros)
    h2, z2, *tabs = _update(S4, z1, h1, W[1], b2[1], g2[1], be2[1])
    S4 = _sc_segsum(tabs, src, dst, zeros)
    return _final(S4, z2, h2, W[2], b2[2], g2[2], be2[2])
